# baseline (speedup 1.0000x reference)
import functools

import jax
import jax.numpy as jnp
from jax import lax
from jax.experimental import pallas as pl
from jax.experimental.pallas import tpu as pltpu
from jax.experimental.pallas import tpu_sc as plsc

_NEG_INF = -1e30
_EPS = 1e-5


def _dot(a, b):
    return jax.lax.dot_general(a, b, (((1,), (0,)), ((), ())),
                               preferred_element_type=jnp.float32)


def _dot_t(a, b):
    return jax.lax.dot_general(a, b, (((1,), (1,)), ((), ())),
                               preferred_element_type=jnp.float32)


def _pq_body(x_ref, w0a_ref, w0d_ref, p_ref, q_ref):
    xb = x_ref[0]
    p_ref[0] = _dot(xb, w0a_ref[...])
    q_ref[0] = _dot(xb, w0d_ref[...])


def _topk_body(K, x_tile_ref, x_full_ref, sqb_ref, idx_ref):
    xr = x_tile_ref[0]
    xb = x_full_ref[0]
    N = xb.shape[0]
    TM = xr.shape[0]

    sqr = jnp.sum(xr * xr, axis=1, keepdims=True)
    sqb = sqb_ref[0]
    inner = _dot_t(xr, xb)
    D = 2.0 * inner - sqr - sqb

    iota = jax.lax.broadcasted_iota(jnp.int32, (TM, N), 1)
    base = pl.program_id(0) * N
    cols = []
    for _ in range(K):
        rm = jnp.max(D, axis=1, keepdims=True)
        M = D >= rm
        ai = jnp.min(jnp.where(M, iota, jnp.int32(N)), axis=1, keepdims=True)
        cols.append(ai)
        D = jnp.where(M, _NEG_INF, D)
    idx_ref[0] = jnp.concatenate(cols, axis=1) + base


def _make_sc_gather(NI, CO, BN):
    info = plsc.get_sparse_core_info()
    NC, NS = info.num_cores, info.num_subcores
    NW = NC * NS
    per_w = NI // NW
    CH = min(512, per_w)
    n_ch = per_w // CH
    mesh = plsc.VectorSubcoreMesh(core_axis_name="c", subcore_axis_name="s")

    @functools.partial(
        pl.kernel, mesh=mesh,
        out_type=jax.ShapeDtypeStruct((NI, CO), jnp.float32),
        scratch_types=[
            pltpu.VMEM((CH,), jnp.int32),
            pltpu.VMEM((CH, CO), jnp.float32),
            pltpu.SemaphoreType.DMA,
        ],
        compiler_params=pltpu.CompilerParams(use_tc_tiling_on_sc=False),
    )
    def sc_gather(table_hbm, idx_hbm, out_hbm, idx_v, rows_v, sem):
        wid = lax.axis_index("s") * NC + lax.axis_index("c")
        w_base = wid * per_w

        def body(g, carry):
            base = pl.multiple_of(w_base + g * CH, 8)
            pltpu.sync_copy(idx_hbm.at[pl.ds(base, CH)], idx_v)
            pltpu.async_copy(table_hbm.at[idx_v], rows_v, sem).wait()
            pltpu.sync_copy(rows_v, out_hbm.at[pl.ds(base, CH)])
            return carry

        lax.fori_loop(0, n_ch, body, 0)

    return sc_gather


def _stats0_body(K, g_ref, q_ref, s0_ref, ss0_ref):
    qr = q_ref[0]
    qt = jnp.concatenate([qr] * K, axis=1)
    h0 = g_ref[0] + qt
    CO = qr.shape[1]
    s = jnp.zeros((CO,), jnp.float32)
    ss = jnp.zeros((CO,), jnp.float32)
    for k in range(K):
        hk = h0[:, k * CO:(k + 1) * CO]
        s = s + jnp.sum(hk, axis=0)
        ss = ss + jnp.sum(hk * hk, axis=0)

    first = (pl.program_id(0) == 0) & (pl.program_id(1) == 0)

    @pl.when(first)
    def _init():
        s0_ref[...] = jnp.zeros_like(s0_ref)
        ss0_ref[...] = jnp.zeros_like(ss0_ref)

    s0_ref[...] += jnp.broadcast_to(s[None, :], s0_ref.shape)
    ss0_ref[...] += jnp.broadcast_to(ss[None, :], ss0_ref.shape)


def _pass2_body(K, g_ref, q_ref, a0_ref, c0_ref, w1_ref,
                mx_ref, mn_ref, s1_ref, ss1_ref):
    qr = q_ref[0]
    qt = jnp.concatenate([qr] * K, axis=1)
    z = a0_ref[...] * (g_ref[0] + qt) + c0_ref[...]
    f = jnp.where(z >= 0, z, 0.01 * z)
    w1 = w1_ref[...]
    CO = w1.shape[0]

    s_acc = jnp.zeros((CO,), jnp.float32)
    ss_acc = jnp.zeros((CO,), jnp.float32)
    mx = None
    mn = None
    for k in range(K):
        fk = f[:, k * CO:(k + 1) * CO]
        hk = _dot(fk, w1)
        s_acc = s_acc + jnp.sum(hk, axis=0)
        ss_acc = ss_acc + jnp.sum(hk * hk, axis=0)
        mx = hk if mx is None else jnp.maximum(mx, hk)
        mn = hk if mn is None else jnp.minimum(mn, hk)

    mx_ref[0] = mx
    mn_ref[0] = mn

    first = (pl.program_id(0) == 0) & (pl.program_id(1) == 0)

    @pl.when(first)
    def _init():
        s1_ref[...] = jnp.zeros_like(s1_ref)
        ss1_ref[...] = jnp.zeros_like(ss1_ref)

    s1_ref[...] += jnp.broadcast_to(s_acc[None, :], s1_ref.shape)
    ss1_ref[...] += jnp.broadcast_to(ss_acc[None, :], ss1_ref.shape)


def _pass3_body(mx_ref, mn_ref, a1_ref, c1_ref, out_ref):
    a1 = a1_ref[...]
    c1 = c1_ref[...]
    pooled = jnp.where(a1 >= 0.0, mx_ref[0], mn_ref[0])
    z = a1 * pooled + c1
    out_ref[0] = jnp.where(z >= 0, z, 0.01 * z)


def kernel(x, W0, gamma0, beta0, W1, gamma1, beta1):
    B, N, C = x.shape
    CO = W0.shape[1]
    K = 32 if N >= 32 else N
    TM = 256 if N % 256 == 0 else N
    nt = N // TM
    cnt = float(B * N * K)
    NI = B * N * K

    W0a = W0[:C]
    W0d = W0[C:] - W0[:C]

    P, Q = pl.pallas_call(
        _pq_body,
        grid=(B,),
        in_specs=[
            pl.BlockSpec((1, N, C), lambda b: (b, 0, 0)),
            pl.BlockSpec((C, CO), lambda b: (0, 0)),
            pl.BlockSpec((C, CO), lambda b: (0, 0)),
        ],
        out_specs=[
            pl.BlockSpec((1, N, CO), lambda b: (b, 0, 0)),
            pl.BlockSpec((1, N, CO), lambda b: (b, 0, 0)),
        ],
        out_shape=[
            jax.ShapeDtypeStruct((B, N, CO), jnp.float32),
            jax.ShapeDtypeStruct((B, N, CO), jnp.float32),
        ],
    )(x, W0a, W0d)

    idx = pl.pallas_call(
        functools.partial(_topk_body, K),
        grid=(B, nt),
        in_specs=[
            pl.BlockSpec((1, TM, C), lambda b, t: (b, t, 0)),
            pl.BlockSpec((1, N, C), lambda b, t: (b, 0, 0)),
            pl.BlockSpec((1, 1, N), lambda b, t: (b, 0, 0)),
        ],
        out_specs=pl.BlockSpec((1, TM, K), lambda b, t: (b, t, 0)),
        out_shape=jax.ShapeDtypeStruct((B, N, K), jnp.int32),
    )(x, x, jnp.sum(x * x, axis=-1).reshape(B, 1, N))

    gather = _make_sc_gather(NI, CO, B * N)
    G = gather(P.reshape(B * N, CO), idx.reshape(NI))
    G = G.reshape(B, N, K * CO)

    s0, ss0 = pl.pallas_call(
        functools.partial(_stats0_body, K),
        grid=(B, nt),
        in_specs=[
            pl.BlockSpec((1, TM, K * CO), lambda b, t: (b, t, 0)),
            pl.BlockSpec((1, TM, CO), lambda b, t: (b, t, 0)),
        ],
        out_specs=[
            pl.BlockSpec((8, CO), lambda b, t: (0, 0)),
            pl.BlockSpec((8, CO), lambda b, t: (0, 0)),
        ],
        out_shape=[
            jax.ShapeDtypeStruct((8, CO), jnp.float32),
            jax.ShapeDtypeStruct((8, CO), jnp.float32),
        ],
    )(G, Q)

    mean0 = s0[0] / cnt
    var0 = ss0[0] / cnt - mean0 * mean0
    a0 = gamma0 / jnp.sqrt(var0 + _EPS)
    c0 = beta0 - a0 * mean0
    a0t = jnp.tile(a0, K)[None, :]
    c0t = jnp.tile(c0, K)[None, :]

    mx, mn, s1, ss1 = pl.pallas_call(
        functools.partial(_pass2_body, K),
        grid=(B, nt),
        in_specs=[
            pl.BlockSpec((1, TM, K * CO), lambda b, t: (b, t, 0)),
            pl.BlockSpec((1, TM, CO), lambda b, t: (b, t, 0)),
            pl.BlockSpec((1, K * CO), lambda b, t: (0, 0)),
            pl.BlockSpec((1, K * CO), lambda b, t: (0, 0)),
            pl.BlockSpec((CO, CO), lambda b, t: (0, 0)),
        ],
        out_specs=[
            pl.BlockSpec((1, TM, CO), lambda b, t: (b, t, 0)),
            pl.BlockSpec((1, TM, CO), lambda b, t: (b, t, 0)),
            pl.BlockSpec((8, CO), lambda b, t: (0, 0)),
            pl.BlockSpec((8, CO), lambda b, t: (0, 0)),
        ],
        out_shape=[
            jax.ShapeDtypeStruct((B, N, CO), jnp.float32),
            jax.ShapeDtypeStruct((B, N, CO), jnp.float32),
            jax.ShapeDtypeStruct((8, CO), jnp.float32),
            jax.ShapeDtypeStruct((8, CO), jnp.float32),
        ],
    )(G, Q, a0t, c0t, W1)

    mean1 = s1[0] / cnt
    var1 = ss1[0] / cnt - mean1 * mean1
    a1 = gamma1 / jnp.sqrt(var1 + _EPS)
    c1 = beta1 - a1 * mean1

    out = pl.pallas_call(
        _pass3_body,
        grid=(B,),
        in_specs=[
            pl.BlockSpec((1, N, CO), lambda b: (b, 0, 0)),
            pl.BlockSpec((1, N, CO), lambda b: (b, 0, 0)),
            pl.BlockSpec((1, CO), lambda b: (0, 0)),
            pl.BlockSpec((1, CO), lambda b: (0, 0)),
        ],
        out_specs=pl.BlockSpec((1, N, CO), lambda b: (b, 0, 0)),
        out_shape=jax.ShapeDtypeStruct((B, N, CO), jnp.float32),
    )(mx, mn, a1[None, :], c1[None, :])

    return out


# speedup vs baseline: 1.0866x; 1.0866x over previous
import functools

import jax
import jax.numpy as jnp
from jax import lax
from jax.experimental import pallas as pl
from jax.experimental.pallas import tpu as pltpu
from jax.experimental.pallas import tpu_sc as plsc

_NEG_INF = -1e30
_EPS = 1e-5


def _dot(a, b):
    return jax.lax.dot_general(a, b, (((1,), (0,)), ((), ())),
                               preferred_element_type=jnp.float32)


def _dot_t(a, b):
    return jax.lax.dot_general(a, b, (((1,), (1,)), ((), ())),
                               preferred_element_type=jnp.float32)


def _pq_body(x_ref, w0a_ref, w0d_ref, p_ref, q_ref):
    xb = x_ref[0]
    p_ref[0] = _dot(xb, w0a_ref[...])
    q_ref[0] = _dot(xb, w0d_ref[...])


def _topk_body(K, x_tile_ref, x_full_ref, sqb_ref, idx_ref):
    xr = x_tile_ref[0]
    xb = x_full_ref[0]
    N = xb.shape[0]
    TM = xr.shape[0]

    sqr = jnp.sum(xr * xr, axis=1, keepdims=True)
    sqb = sqb_ref[0]
    inner = _dot_t(xr, xb)
    D = 2.0 * inner - sqr - sqb

    iota_col = jax.lax.broadcasted_iota(jnp.int32, (N, 1), 0).astype(jnp.float32)
    base = pl.program_id(0) * N
    cols = []
    for _ in range(K):
        rm = jnp.max(D, axis=1, keepdims=True)
        M = D >= rm
        Mf = M.astype(jnp.float32)
        cols.append(_dot(Mf, iota_col))
        D = jnp.where(M, _NEG_INF, D)
    idx_ref[0] = jnp.concatenate(cols, axis=1).astype(jnp.int32) + base


def _make_sc_gather(NI, CO, BN):
    info = plsc.get_sparse_core_info()
    NC, NS = info.num_cores, info.num_subcores
    NW = NC * NS
    per_w = NI // NW
    CH = min(512, per_w)
    n_ch = per_w // CH
    mesh = plsc.VectorSubcoreMesh(core_axis_name="c", subcore_axis_name="s")

    @functools.partial(
        pl.kernel, mesh=mesh,
        out_type=jax.ShapeDtypeStruct((NI, CO), jnp.float32),
        scratch_types=[
            pltpu.VMEM((2, CH), jnp.int32),
            pltpu.VMEM((2, CH, CO), jnp.float32),
            pltpu.SemaphoreType.DMA,
            pltpu.SemaphoreType.DMA,
            pltpu.SemaphoreType.DMA,
        ],
        compiler_params=pltpu.CompilerParams(use_tc_tiling_on_sc=False),
    )
    def sc_gather(table_hbm, idx_hbm, out_hbm, idx_v, rows_v, sem_g,
                  sem_s0, sem_s1):
        wid = lax.axis_index("s") * NC + lax.axis_index("c")
        w_base = wid * per_w
        sems = (sem_s0, sem_s1)

        def body(h, carry):
            for par in range(2):
                g = h * 2 + par
                base = pl.multiple_of(w_base + g * CH, 8)
                pltpu.sync_copy(idx_hbm.at[pl.ds(base, CH)], idx_v.at[par])

                @pl.when(h >= 1)
                def _drain():
                    prev = pl.multiple_of(base - 2 * CH, 8)
                    pltpu.make_async_copy(
                        rows_v.at[par], out_hbm.at[pl.ds(prev, CH)],
                        sems[par]).wait()

                pltpu.async_copy(table_hbm.at[idx_v.at[par]],
                                 rows_v.at[par], sem_g).wait()
                pltpu.async_copy(rows_v.at[par],
                                 out_hbm.at[pl.ds(base, CH)], sems[par])
            return carry

        lax.fori_loop(0, n_ch // 2, body, 0)
        for par in range(2):
            last = pl.multiple_of(w_base + (n_ch - 2 + par) * CH, 8)
            pltpu.make_async_copy(rows_v.at[par],
                                  out_hbm.at[pl.ds(last, CH)],
                                  sems[par]).wait()

    return sc_gather


def _stats0_body(K, g_ref, q_ref, s0_ref, ss0_ref):
    qr = q_ref[0]
    qt = jnp.concatenate([qr] * K, axis=1)
    h0 = g_ref[0] + qt
    CO = qr.shape[1]
    s = jnp.zeros((CO,), jnp.float32)
    ss = jnp.zeros((CO,), jnp.float32)
    for k in range(K):
        hk = h0[:, k * CO:(k + 1) * CO]
        s = s + jnp.sum(hk, axis=0)
        ss = ss + jnp.sum(hk * hk, axis=0)

    first = (pl.program_id(0) == 0) & (pl.program_id(1) == 0)

    @pl.when(first)
    def _init():
        s0_ref[...] = jnp.zeros_like(s0_ref)
        ss0_ref[...] = jnp.zeros_like(ss0_ref)

    s0_ref[...] += jnp.broadcast_to(s[None, :], s0_ref.shape)
    ss0_ref[...] += jnp.broadcast_to(ss[None, :], ss0_ref.shape)


def _pass2_body(K, g_ref, q_ref, a0_ref, c0_ref, w1_ref,
                mx_ref, mn_ref, s1_ref, ss1_ref):
    qr = q_ref[0]
    qt = jnp.concatenate([qr] * K, axis=1)
    z = a0_ref[...] * (g_ref[0] + qt) + c0_ref[...]
    f = jnp.where(z >= 0, z, 0.01 * z)
    w1 = w1_ref[...]
    CO = w1.shape[0]

    s_acc = jnp.zeros((CO,), jnp.float32)
    ss_acc = jnp.zeros((CO,), jnp.float32)
    mx = None
    mn = None
    for k in range(K):
        fk = f[:, k * CO:(k + 1) * CO]
        hk = _dot(fk, w1)
        s_acc = s_acc + jnp.sum(hk, axis=0)
        ss_acc = ss_acc + jnp.sum(hk * hk, axis=0)
        mx = hk if mx is None else jnp.maximum(mx, hk)
        mn = hk if mn is None else jnp.minimum(mn, hk)

    mx_ref[0] = mx
    mn_ref[0] = mn

    first = (pl.program_id(0) == 0) & (pl.program_id(1) == 0)

    @pl.when(first)
    def _init():
        s1_ref[...] = jnp.zeros_like(s1_ref)
        ss1_ref[...] = jnp.zeros_like(ss1_ref)

    s1_ref[...] += jnp.broadcast_to(s_acc[None, :], s1_ref.shape)
    ss1_ref[...] += jnp.broadcast_to(ss_acc[None, :], ss1_ref.shape)


def _pass3_body(mx_ref, mn_ref, a1_ref, c1_ref, out_ref):
    a1 = a1_ref[...]
    c1 = c1_ref[...]
    pooled = jnp.where(a1 >= 0.0, mx_ref[0], mn_ref[0])
    z = a1 * pooled + c1
    out_ref[0] = jnp.where(z >= 0, z, 0.01 * z)


def kernel(x, W0, gamma0, beta0, W1, gamma1, beta1):
    B, N, C = x.shape
    CO = W0.shape[1]
    K = 32 if N >= 32 else N
    TM = 256 if N % 256 == 0 else N
    nt = N // TM
    cnt = float(B * N * K)
    NI = B * N * K

    W0a = W0[:C]
    W0d = W0[C:] - W0[:C]

    P, Q = pl.pallas_call(
        _pq_body,
        grid=(B,),
        in_specs=[
            pl.BlockSpec((1, N, C), lambda b: (b, 0, 0)),
            pl.BlockSpec((C, CO), lambda b: (0, 0)),
            pl.BlockSpec((C, CO), lambda b: (0, 0)),
        ],
        out_specs=[
            pl.BlockSpec((1, N, CO), lambda b: (b, 0, 0)),
            pl.BlockSpec((1, N, CO), lambda b: (b, 0, 0)),
        ],
        out_shape=[
            jax.ShapeDtypeStruct((B, N, CO), jnp.float32),
            jax.ShapeDtypeStruct((B, N, CO), jnp.float32),
        ],
    )(x, W0a, W0d)

    idx = pl.pallas_call(
        functools.partial(_topk_body, K),
        grid=(B, nt),
        in_specs=[
            pl.BlockSpec((1, TM, C), lambda b, t: (b, t, 0)),
            pl.BlockSpec((1, N, C), lambda b, t: (b, 0, 0)),
            pl.BlockSpec((1, 1, N), lambda b, t: (b, 0, 0)),
        ],
        out_specs=pl.BlockSpec((1, TM, K), lambda b, t: (b, t, 0)),
        out_shape=jax.ShapeDtypeStruct((B, N, K), jnp.int32),
    )(x, x, jnp.sum(x * x, axis=-1).reshape(B, 1, N))

    gather = _make_sc_gather(NI, CO, B * N)
    G = gather(P.reshape(B * N, CO), idx.reshape(NI))
    G = G.reshape(B, N, K * CO)

    s0, ss0 = pl.pallas_call(
        functools.partial(_stats0_body, K),
        grid=(B, nt),
        in_specs=[
            pl.BlockSpec((1, TM, K * CO), lambda b, t: (b, t, 0)),
            pl.BlockSpec((1, TM, CO), lambda b, t: (b, t, 0)),
        ],
        out_specs=[
            pl.BlockSpec((8, CO), lambda b, t: (0, 0)),
            pl.BlockSpec((8, CO), lambda b, t: (0, 0)),
        ],
        out_shape=[
            jax.ShapeDtypeStruct((8, CO), jnp.float32),
            jax.ShapeDtypeStruct((8, CO), jnp.float32),
        ],
    )(G, Q)

    mean0 = s0[0] / cnt
    var0 = ss0[0] / cnt - mean0 * mean0
    a0 = gamma0 / jnp.sqrt(var0 + _EPS)
    c0 = beta0 - a0 * mean0
    a0t = jnp.tile(a0, K)[None, :]
    c0t = jnp.tile(c0, K)[None, :]

    mx, mn, s1, ss1 = pl.pallas_call(
        functools.partial(_pass2_body, K),
        grid=(B, nt),
        in_specs=[
            pl.BlockSpec((1, TM, K * CO), lambda b, t: (b, t, 0)),
            pl.BlockSpec((1, TM, CO), lambda b, t: (b, t, 0)),
            pl.BlockSpec((1, K * CO), lambda b, t: (0, 0)),
            pl.BlockSpec((1, K * CO), lambda b, t: (0, 0)),
            pl.BlockSpec((CO, CO), lambda b, t: (0, 0)),
        ],
        out_specs=[
            pl.BlockSpec((1, TM, CO), lambda b, t: (b, t, 0)),
            pl.BlockSpec((1, TM, CO), lambda b, t: (b, t, 0)),
            pl.BlockSpec((8, CO), lambda b, t: (0, 0)),
            pl.BlockSpec((8, CO), lambda b, t: (0, 0)),
        ],
        out_shape=[
            jax.ShapeDtypeStruct((B, N, CO), jnp.float32),
            jax.ShapeDtypeStruct((B, N, CO), jnp.float32),
            jax.ShapeDtypeStruct((8, CO), jnp.float32),
            jax.ShapeDtypeStruct((8, CO), jnp.float32),
        ],
    )(G, Q, a0t, c0t, W1)

    mean1 = s1[0] / cnt
    var1 = ss1[0] / cnt - mean1 * mean1
    a1 = gamma1 / jnp.sqrt(var1 + _EPS)
    c1 = beta1 - a1 * mean1

    out = pl.pallas_call(
        _pass3_body,
        grid=(B,),
        in_specs=[
            pl.BlockSpec((1, N, CO), lambda b: (b, 0, 0)),
            pl.BlockSpec((1, N, CO), lambda b: (b, 0, 0)),
            pl.BlockSpec((1, CO), lambda b: (0, 0)),
            pl.BlockSpec((1, CO), lambda b: (0, 0)),
        ],
        out_specs=pl.BlockSpec((1, N, CO), lambda b: (b, 0, 0)),
        out_shape=jax.ShapeDtypeStruct((B, N, CO), jnp.float32),
    )(mx, mn, a1[None, :], c1[None, :])

    return out


# speedup vs baseline: 1.4469x; 1.3316x over previous
import functools

import jax
import jax.numpy as jnp
from jax.experimental import pallas as pl

_NEG_INF = -1e30
_EPS = 1e-5


def _dot(a, b):
    return jax.lax.dot_general(a, b, (((1,), (0,)), ((), ())),
                               preferred_element_type=jnp.float32)


def _dot_t(a, b):
    return jax.lax.dot_general(a, b, (((1,), (1,)), ((), ())),
                               preferred_element_type=jnp.float32)


def _pass1_body(K, x_tile_ref, x_full_ref, sqb_ref, w0a_ref, w0d_ref,
                h0_ref, s0_ref, ss0_ref):
    xr = x_tile_ref[0]
    xb = x_full_ref[0]
    w0a = w0a_ref[...]
    w0d = w0d_ref[...]

    P = _dot(xb, w0a)
    Qr = _dot(xr, w0d)

    sqr = jnp.sum(xr * xr, axis=1, keepdims=True)
    sqb = sqb_ref[0]
    inner = _dot_t(xr, xb)
    D = 2.0 * inner - sqr - sqb

    CO = w0a.shape[1]
    s_acc = jnp.zeros((CO,), jnp.float32)
    ss_acc = jnp.zeros((CO,), jnp.float32)
    for k in range(K):
        rm = jnp.max(D, axis=1, keepdims=True)
        M = D >= rm
        G = _dot(M.astype(jnp.float32), P)
        h0t = G + Qr
        h0_ref[0, :, k * CO:(k + 1) * CO] = h0t
        s_acc = s_acc + jnp.sum(h0t, axis=0)
        ss_acc = ss_acc + jnp.sum(h0t * h0t, axis=0)
        D = jnp.where(M, _NEG_INF, D)

    first = (pl.program_id(0) == 0) & (pl.program_id(1) == 0)

    @pl.when(first)
    def _init():
        s0_ref[...] = jnp.zeros_like(s0_ref)
        ss0_ref[...] = jnp.zeros_like(ss0_ref)

    s0_ref[...] += jnp.broadcast_to(s_acc[None, :], s0_ref.shape)
    ss0_ref[...] += jnp.broadcast_to(ss_acc[None, :], ss0_ref.shape)


def _pass2_body(K, h0_ref, a0_ref, c0_ref, w1_ref,
                mx_ref, mn_ref, s1_ref, ss1_ref):
    z = a0_ref[...] * h0_ref[0] + c0_ref[...]
    f = jnp.where(z >= 0, z, 0.01 * z)
    w1 = w1_ref[...]
    CO = w1.shape[0]

    s_acc = jnp.zeros((CO,), jnp.float32)
    ss_acc = jnp.zeros((CO,), jnp.float32)
    mx = None
    mn = None
    for k in range(K):
        fk = f[:, k * CO:(k + 1) * CO]
        hk = _dot(fk, w1)
        s_acc = s_acc + jnp.sum(hk, axis=0)
        ss_acc = ss_acc + jnp.sum(hk * hk, axis=0)
        mx = hk if mx is None else jnp.maximum(mx, hk)
        mn = hk if mn is None else jnp.minimum(mn, hk)

    mx_ref[0] = mx
    mn_ref[0] = mn

    first = (pl.program_id(0) == 0) & (pl.program_id(1) == 0)

    @pl.when(first)
    def _init():
        s1_ref[...] = jnp.zeros_like(s1_ref)
        ss1_ref[...] = jnp.zeros_like(ss1_ref)

    s1_ref[...] += jnp.broadcast_to(s_acc[None, :], s1_ref.shape)
    ss1_ref[...] += jnp.broadcast_to(ss_acc[None, :], ss1_ref.shape)


def _pass3_body(mx_ref, mn_ref, a1_ref, c1_ref, out_ref):
    a1 = a1_ref[...]
    c1 = c1_ref[...]
    pooled = jnp.where(a1 >= 0.0, mx_ref[0], mn_ref[0])
    z = a1 * pooled + c1
    out_ref[0] = jnp.where(z >= 0, z, 0.01 * z)


def kernel(x, W0, gamma0, beta0, W1, gamma1, beta1):
    B, N, C = x.shape
    CO = W0.shape[1]
    K = 32 if N >= 32 else N
    TM = 256 if N % 256 == 0 else N
    nt = N // TM
    cnt = float(B * N * K)

    W0a = W0[:C]
    W0d = W0[C:] - W0[:C]

    h0, s0, ss0 = pl.pallas_call(
        functools.partial(_pass1_body, K),
        grid=(B, nt),
        in_specs=[
            pl.BlockSpec((1, TM, C), lambda b, t: (b, t, 0)),
            pl.BlockSpec((1, N, C), lambda b, t: (b, 0, 0)),
            pl.BlockSpec((1, 1, N), lambda b, t: (b, 0, 0)),
            pl.BlockSpec((C, CO), lambda b, t: (0, 0)),
            pl.BlockSpec((C, CO), lambda b, t: (0, 0)),
        ],
        out_specs=[
            pl.BlockSpec((1, TM, K * CO), lambda b, t: (b, t, 0)),
            pl.BlockSpec((8, CO), lambda b, t: (0, 0)),
            pl.BlockSpec((8, CO), lambda b, t: (0, 0)),
        ],
        out_shape=[
            jax.ShapeDtypeStruct((B, N, K * CO), jnp.float32),
            jax.ShapeDtypeStruct((8, CO), jnp.float32),
            jax.ShapeDtypeStruct((8, CO), jnp.float32),
        ],
    )(x, x, jnp.sum(x * x, axis=-1).reshape(B, 1, N), W0a, W0d)

    mean0 = s0[0] / cnt
    var0 = ss0[0] / cnt - mean0 * mean0
    a0 = gamma0 / jnp.sqrt(var0 + _EPS)
    c0 = beta0 - a0 * mean0
    a0t = jnp.tile(a0, K)[None, :]
    c0t = jnp.tile(c0, K)[None, :]

    mx, mn, s1, ss1 = pl.pallas_call(
        functools.partial(_pass2_body, K),
        grid=(B, nt),
        in_specs=[
            pl.BlockSpec((1, TM, K * CO), lambda b, t: (b, t, 0)),
            pl.BlockSpec((1, K * CO), lambda b, t: (0, 0)),
            pl.BlockSpec((1, K * CO), lambda b, t: (0, 0)),
            pl.BlockSpec((CO, CO), lambda b, t: (0, 0)),
        ],
        out_specs=[
            pl.BlockSpec((1, TM, CO), lambda b, t: (b, t, 0)),
            pl.BlockSpec((1, TM, CO), lambda b, t: (b, t, 0)),
            pl.BlockSpec((8, CO), lambda b, t: (0, 0)),
            pl.BlockSpec((8, CO), lambda b, t: (0, 0)),
        ],
        out_shape=[
            jax.ShapeDtypeStruct((B, N, CO), jnp.float32),
            jax.ShapeDtypeStruct((B, N, CO), jnp.float32),
            jax.ShapeDtypeStruct((8, CO), jnp.float32),
            jax.ShapeDtypeStruct((8, CO), jnp.float32),
        ],
    )(h0, a0t, c0t, W1)

    mean1 = s1[0] / cnt
    var1 = ss1[0] / cnt - mean1 * mean1
    a1 = gamma1 / jnp.sqrt(var1 + _EPS)
    c1 = beta1 - a1 * mean1

    out = pl.pallas_call(
        _pass3_body,
        grid=(B,),
        in_specs=[
            pl.BlockSpec((1, N, CO), lambda b: (b, 0, 0)),
            pl.BlockSpec((1, N, CO), lambda b: (b, 0, 0)),
            pl.BlockSpec((1, CO), lambda b: (0, 0)),
            pl.BlockSpec((1, CO), lambda b: (0, 0)),
        ],
        out_specs=pl.BlockSpec((1, N, CO), lambda b: (b, 0, 0)),
        out_shape=jax.ShapeDtypeStruct((B, N, CO), jnp.float32),
    )(mx, mn, a1[None, :], c1[None, :])

    return out
